# Initial kernel scaffold; baseline (speedup 1.0000x reference)
import jax
import jax.numpy as jnp
from jax.experimental import pallas as pl


def kernel(x, edge_index, batch, W1, b1, W2, b2, W3, b3, ln1_g, ln1_b, ln2_g, ln2_b, Wp1, bp1, Wp2, bp2):
    raise NotImplementedError("write your pallas kernel here")


# speedup vs baseline: 13.2989x; 13.2989x over previous
import functools

import jax
import jax.numpy as jnp
from jax import lax
from jax.experimental import pallas as pl
from jax.experimental.pallas import tpu as pltpu
from jax.experimental.pallas import tpu_sc as plsc

N = 10000
E = 320000
D = 128
N_PAD = 10240
NC, NS = 2, 16
NW = NC * NS
CHUNK = 128
R = ((E + NW * CHUNK - 1) // (NW * CHUNK)) * NW
E_PAD = R * CHUNK
CPT = R // NW
ROWS_PT = N_PAD // NS

@functools.cache
def _mesh():
    return plsc.VectorSubcoreMesh(
        core_axis_name="c", subcore_axis_name="s",
        num_cores=NC, num_subcores=NS)


EPT = CPT * CHUNK


def _sc_deg(dst1):
    @functools.partial(
        pl.kernel,
        mesh=_mesh(),
        out_type=jax.ShapeDtypeStruct((NC * N_PAD,), jnp.float32),
        scratch_types=[
            pltpu.VMEM((CHUNK,), jnp.int32),
            pltpu.VMEM((CHUNK,), jnp.float32),
            pltpu.VMEM((ROWS_PT,), jnp.float32),
            pltpu.VMEM_SHARED((N_PAD,), jnp.float32),
        ],
    )
    def k(dst_hbm, out_hbm, idx_v, ones_v, zbuf_v, acc_sh):
        c = lax.axis_index("c")
        s = lax.axis_index("s")
        wid = c * NS + s

        @pl.loop(0, CHUNK // 16)
        def _(i):
            ones_v[pl.ds(i * 16, 16)] = jnp.ones((16,), jnp.float32)

        @pl.loop(0, ROWS_PT // 16)
        def _(i):
            zbuf_v[pl.ds(i * 16, 16)] = jnp.zeros((16,), jnp.float32)

        pltpu.sync_copy(zbuf_v, acc_sh.at[pl.ds(s * ROWS_PT, ROWS_PT)])
        plsc.subcore_barrier()

        base = wid * EPT

        @pl.loop(0, CPT)
        def _(j):
            pltpu.sync_copy(dst_hbm.at[pl.ds(base + j * CHUNK, CHUNK)], idx_v)
            pltpu.sync_copy(ones_v, acc_sh.at[idx_v], add=True)

        plsc.subcore_barrier()
        pltpu.sync_copy(acc_sh.at[pl.ds(s * ROWS_PT, ROWS_PT)],
                        out_hbm.at[pl.ds(c * N_PAD + s * ROWS_PT, ROWS_PT)])

    return k(dst1)


def _sc_scatter(g, src1, dst1, zD):
    @functools.partial(
        pl.kernel,
        mesh=_mesh(),
        out_type=jax.ShapeDtypeStruct((NC * N_PAD, D), jnp.float32),
        scratch_types=[
            pltpu.VMEM((CHUNK,), jnp.int32),
            pltpu.VMEM((CHUNK,), jnp.int32),
            pltpu.VMEM((CHUNK, D), jnp.float32),
            pltpu.VMEM_SHARED((N_PAD, D), jnp.float32),
            pltpu.SemaphoreType.DMA,
        ],
    )
    def k(g_hbm, src_hbm, dst_hbm, z_hbm, out_hbm, is_v, id_v, rows_v,
          acc_sh, sem):
        c = lax.axis_index("c")
        s = lax.axis_index("s")
        wid = c * NS + s

        @pl.when(s == 0)
        def _():
            pltpu.sync_copy(z_hbm, acc_sh)

        plsc.subcore_barrier()

        base = wid * EPT

        @pl.loop(0, CPT)
        def _(j):
            pltpu.sync_copy(src_hbm.at[pl.ds(base + j * CHUNK, CHUNK)], is_v)
            pltpu.sync_copy(dst_hbm.at[pl.ds(base + j * CHUNK, CHUNK)], id_v)
            pltpu.async_copy(g_hbm.at[is_v], rows_v, sem).wait()
            pltpu.sync_copy(rows_v, acc_sh.at[id_v], add=True)

        plsc.subcore_barrier()
        pltpu.sync_copy(acc_sh.at[pl.ds(s * ROWS_PT, ROWS_PT)],
                        out_hbm.at[pl.ds(c * N_PAD + s * ROWS_PT, ROWS_PT)])

    return k(g, src1, dst1, zD)


_BN = 1024


def _tc_dinv(degp):
    def body(dp_ref, o_ref):
        i = pl.program_id(0)
        d = dp_ref[0] + dp_ref[1] + 1.0
        row = lax.broadcasted_iota(jnp.int32, (_BN, 1), 0) + i * _BN
        o_ref[...] = jnp.where(row < N, lax.rsqrt(d), 0.0)

    return pl.pallas_call(
        body,
        grid=(N_PAD // _BN,),
        in_specs=[pl.BlockSpec((NC, _BN, 1), lambda i: (0, i, 0))],
        out_specs=pl.BlockSpec((_BN, 1), lambda i: (i, 0)),
        out_shape=jax.ShapeDtypeStruct((N_PAD, 1), jnp.float32),
    )(degp)


def _tc_pre(y, W, dinv):
    def body(y_ref, w_ref, d_ref, o_ref):
        h = jnp.dot(y_ref[...], w_ref[...], preferred_element_type=jnp.float32)
        o_ref[...] = h * d_ref[...]

    return pl.pallas_call(
        body,
        grid=(N_PAD // _BN,),
        in_specs=[
            pl.BlockSpec((_BN, D), lambda i: (i, 0)),
            pl.BlockSpec((D, D), lambda i: (0, 0)),
            pl.BlockSpec((_BN, 1), lambda i: (i, 0)),
        ],
        out_specs=pl.BlockSpec((_BN, D), lambda i: (i, 0)),
        out_shape=jax.ShapeDtypeStruct((N_PAD, D), jnp.float32),
    )(y, W, dinv)


def _combine(p_ref, g_ref, d_ref, b_ref):
    t = (p_ref[0] + p_ref[1] + g_ref[...]) * d_ref[...] + b_ref[...]
    return jnp.maximum(t, 0.0)


def _tc_post(parts, g, dinv, b, ln_g, ln_b):
    def body(p_ref, g_ref, d_ref, b_ref, lg_ref, lb_ref, o_ref):
        r = _combine(p_ref, g_ref, d_ref, b_ref)
        mu = jnp.mean(r, axis=-1, keepdims=True)
        var = jnp.mean((r - mu) ** 2, axis=-1, keepdims=True)
        o_ref[...] = (r - mu) * lax.rsqrt(var + 1e-5) * lg_ref[...] + lb_ref[...]

    return pl.pallas_call(
        body,
        grid=(N_PAD // _BN,),
        in_specs=[
            pl.BlockSpec((NC, _BN, D), lambda i: (0, i, 0)),
            pl.BlockSpec((_BN, D), lambda i: (i, 0)),
            pl.BlockSpec((_BN, 1), lambda i: (i, 0)),
            pl.BlockSpec((1, D), lambda i: (0, 0)),
            pl.BlockSpec((1, D), lambda i: (0, 0)),
            pl.BlockSpec((1, D), lambda i: (0, 0)),
        ],
        out_specs=pl.BlockSpec((_BN, D), lambda i: (i, 0)),
        out_shape=jax.ShapeDtypeStruct((N_PAD, D), jnp.float32),
    )(parts, g, dinv, b.reshape(1, D), ln_g.reshape(1, D), ln_b.reshape(1, D))


def _tc_head(parts, g, dinv, b, Wp1, bp1, Wp2, bp2):
    def body(p_ref, g_ref, d_ref, b_ref, w1_ref, b1_ref, w2_ref, b2_ref,
             o_ref):
        r = _combine(p_ref, g_ref, d_ref, b_ref)
        t = jnp.dot(r, w1_ref[...], preferred_element_type=jnp.float32)
        t = t + b1_ref[...]
        t = jnp.dot(t, w2_ref[...], preferred_element_type=jnp.float32)
        o_ref[...] = t + b2_ref[...]

    return pl.pallas_call(
        body,
        grid=(N_PAD // _BN,),
        in_specs=[
            pl.BlockSpec((NC, _BN, D), lambda i: (0, i, 0)),
            pl.BlockSpec((_BN, D), lambda i: (i, 0)),
            pl.BlockSpec((_BN, 1), lambda i: (i, 0)),
            pl.BlockSpec((1, D), lambda i: (0, 0)),
            pl.BlockSpec((D, D), lambda i: (0, 0)),
            pl.BlockSpec((1, D), lambda i: (0, 0)),
            pl.BlockSpec((D, D), lambda i: (0, 0)),
            pl.BlockSpec((1, D), lambda i: (0, 0)),
        ],
        out_specs=pl.BlockSpec((_BN, D), lambda i: (i, 0)),
        out_shape=jax.ShapeDtypeStruct((N_PAD, D), jnp.float32),
    )(parts, g, dinv, b.reshape(1, D), Wp1, bp1.reshape(1, D), Wp2,
      bp2.reshape(1, D))


def kernel(x, edge_index, batch, W1, b1, W2, b2, W3, b3,
           ln1_g, ln1_b, ln2_g, ln2_b, Wp1, bp1, Wp2, bp2):
    src, dst = edge_index[0], edge_index[1]
    pad = E_PAD - E
    padi = jnp.arange(pad, dtype=src.dtype)
    trash = N + padi % (N_PAD - N)
    src1 = jnp.concatenate([src, trash])
    dst1 = jnp.concatenate([dst, trash])
    x_p = jnp.pad(x, ((0, N_PAD - N), (0, 0)))
    zD = jnp.zeros((N_PAD, D), jnp.float32)

    degp = _sc_deg(dst1).reshape(NC, N_PAD, 1)
    dinv = _tc_dinv(degp)

    g1 = _tc_pre(x_p, W1, dinv)
    p1 = _sc_scatter(g1, src1, dst1, zD).reshape(NC, N_PAD, D)
    y1 = _tc_post(p1, g1, dinv, b1, ln1_g, ln1_b)

    g2 = _tc_pre(y1, W2, dinv)
    p2 = _sc_scatter(g2, src1, dst1, zD).reshape(NC, N_PAD, D)
    y2 = _tc_post(p2, g2, dinv, b2, ln2_g, ln2_b)

    g3 = _tc_pre(y2, W3, dinv)
    p3 = _sc_scatter(g3, src1, dst1, zD).reshape(NC, N_PAD, D)
    out = _tc_head(p3, g3, dinv, b3, Wp1, bp1, Wp2, bp2)
    return out[:N]


# speedup vs baseline: 16.8360x; 1.2660x over previous
import functools

import jax
import jax.numpy as jnp
from jax import lax
from jax.experimental import pallas as pl
from jax.experimental.pallas import tpu as pltpu
from jax.experimental.pallas import tpu_sc as plsc

N = 10000
E = 320000
D = 128
DH = 64
N_PAD = 10240
NC, NS = 2, 16
NW = NC * NS
CHUNK = 128
CPT = 160
E_PAD = NS * CPT * CHUNK
EPT = CPT * CHUNK
CPT_DEG = E_PAD // (NW * CHUNK)
ROWS_PT = N_PAD // NS
_B = 2
_NBATCH = CPT // _B


@functools.cache
def _mesh():
    return plsc.VectorSubcoreMesh(
        core_axis_name="c", subcore_axis_name="s",
        num_cores=NC, num_subcores=NS)


def _sc_deg(dst2):
    @functools.partial(
        pl.kernel,
        mesh=_mesh(),
        out_type=jax.ShapeDtypeStruct((NC * N_PAD,), jnp.float32),
        scratch_types=[
            pltpu.VMEM((CPT_DEG, CHUNK), jnp.int32),
            pltpu.VMEM((CHUNK,), jnp.float32),
            pltpu.VMEM((ROWS_PT,), jnp.float32),
            pltpu.VMEM_SHARED((N_PAD,), jnp.float32),
        ],
    )
    def k(dst_hbm, out_hbm, id2_v, ones_v, zbuf_v, acc_sh):
        c = lax.axis_index("c")
        s = lax.axis_index("s")
        wid = c * NS + s

        pltpu.sync_copy(dst_hbm.at[pl.ds(wid * CPT_DEG, CPT_DEG)], id2_v)

        @pl.loop(0, CHUNK // 16)
        def _(i):
            ones_v[pl.ds(i * 16, 16)] = jnp.ones((16,), jnp.float32)

        @pl.loop(0, ROWS_PT // 16)
        def _(i):
            zbuf_v[pl.ds(i * 16, 16)] = jnp.zeros((16,), jnp.float32)

        pltpu.sync_copy(zbuf_v, acc_sh.at[pl.ds(s * ROWS_PT, ROWS_PT)])
        plsc.subcore_barrier()

        @pl.loop(0, CPT_DEG)
        def _(j):
            pltpu.sync_copy(ones_v, acc_sh.at[id2_v.at[j]], add=True)

        plsc.subcore_barrier()
        pltpu.sync_copy(acc_sh.at[pl.ds(s * ROWS_PT, ROWS_PT)],
                        out_hbm.at[pl.ds(c * N_PAD + s * ROWS_PT, ROWS_PT)])

    return k(dst2)


def _sc_scatter(g2, srcoff, dst2):
    @functools.partial(
        pl.kernel,
        mesh=_mesh(),
        out_type=jax.ShapeDtypeStruct((NC * N_PAD, DH), jnp.float32),
        compiler_params=pltpu.CompilerParams(use_tc_tiling_on_sc=False),
        scratch_types=[
            pltpu.VMEM((CPT, CHUNK), jnp.int32),
            pltpu.VMEM((_B, CHUNK, DH), jnp.float32),
            pltpu.VMEM((_B, CHUNK, DH), jnp.float32),
            [pltpu.VMEM((CHUNK,), jnp.int32)] * (2 * _B),
            pltpu.VMEM_SHARED((N_PAD, DH), jnp.float32),
            pltpu.SemaphoreType.DMA,
            pltpu.SemaphoreType.DMA,
        ],
    )
    def k(g_hbm, src_hbm, dst_hbm, out_hbm, id2_v, rows_a, rows_b,
          sbufs_flat, acc_sh, semg, sems):
        halves = (rows_a, rows_b)
        sbufs = (sbufs_flat[:_B], sbufs_flat[_B:])
        c = lax.axis_index("c")
        s = lax.axis_index("s")

        pltpu.sync_copy(dst_hbm.at[pl.ds(s * CPT, CPT)], id2_v)

        @pl.loop(0, CHUNK)
        def _(i):
            for l in range(DH // 16):
                rows_a[0, i, pl.ds(l * 16, 16)] = jnp.zeros(
                    (16,), jnp.float32)

        for r in range(ROWS_PT // CHUNK):
            pltpu.sync_copy(
                rows_a.at[0],
                acc_sh.at[pl.ds(s * ROWS_PT + r * CHUNK, CHUNK)])

        plsc.subcore_barrier()

        src_base = c * E_PAD + s * EPT

        def fire_g(t, h):
            for i in range(_B):
                pltpu.sync_copy(
                    src_hbm.at[pl.ds(src_base + (t * _B + i) * CHUNK,
                                     CHUNK)], sbufs[h][i])
                pltpu.async_copy(g_hbm.at[sbufs[h][i]], halves[h].at[i],
                                 semg)

        def drain_g(t, h):
            for i in range(_B):
                pltpu.make_async_copy(
                    g_hbm.at[sbufs[h][i]], halves[h].at[i], semg).wait()

        def fire_s(t, h):
            for i in range(_B):
                pltpu.async_copy(
                    halves[h].at[i], acc_sh.at[id2_v.at[t * _B + i]], sems,
                    add=True)

        def drain_s(t, h):
            for i in range(_B):
                pltpu.make_async_copy(
                    halves[h].at[i], acc_sh.at[id2_v.at[t * _B + i]],
                    sems).wait()

        fire_g(0, 0)

        @pl.loop(0, _NBATCH // 2)
        def _(u):
            for h in range(2):
                t = 2 * u + h
                drain_g(t, h)

                @pl.when(t + 1 < _NBATCH)
                def _():
                    fire_g(t + 1, 1 - h)

                fire_s(t, h)
                drain_s(t, h)

        plsc.subcore_barrier()
        pltpu.sync_copy(acc_sh.at[pl.ds(s * ROWS_PT, ROWS_PT)],
                        out_hbm.at[pl.ds(c * N_PAD + s * ROWS_PT, ROWS_PT)])

    return k(g2, srcoff, dst2)


_BN = 1024
_NB = N_PAD // _BN


def _tc_dinv(degp):
    def body(dp_ref, o_ref):
        i = pl.program_id(0)
        d = dp_ref[0] + dp_ref[1] + 1.0
        row = lax.broadcasted_iota(jnp.int32, (_BN, 1), 0) + i * _BN
        o_ref[...] = jnp.where(row < N, lax.rsqrt(d), 0.0)

    return pl.pallas_call(
        body,
        grid=(_NB,),
        in_specs=[pl.BlockSpec((NC, _BN, 1), lambda i: (0, i, 0))],
        out_specs=pl.BlockSpec((_BN, 1), lambda i: (i, 0)),
        out_shape=jax.ShapeDtypeStruct((N_PAD, 1), jnp.float32),
    )(degp)


def _tc_pre(y, W, dinv):
    Wsp = jnp.stack([W[:, :DH], W[:, DH:]], axis=0)

    def body(y_ref, w_ref, d_ref, o_ref):
        h = jnp.dot(y_ref[...], w_ref[0],
                    preferred_element_type=jnp.float32)
        o_ref[...] = h * d_ref[...]

    return pl.pallas_call(
        body,
        grid=(_NB, NC),
        in_specs=[
            pl.BlockSpec((_BN, D), lambda i, c: (i, 0)),
            pl.BlockSpec((1, D, DH), lambda i, c: (c, 0, 0)),
            pl.BlockSpec((_BN, 1), lambda i, c: (i, 0)),
        ],
        out_specs=pl.BlockSpec((_BN, DH), lambda i, c: (c * _NB + i, 0)),
        out_shape=jax.ShapeDtypeStruct((NC * N_PAD, DH), jnp.float32),
    )(y, Wsp, dinv)


def _combine(p0_ref, p1_ref, g0_ref, g1_ref, d_ref, b_ref):
    p = jnp.concatenate([p0_ref[...] + g0_ref[...],
                         p1_ref[...] + g1_ref[...]], axis=1)
    return jnp.maximum(p * d_ref[...] + b_ref[...], 0.0)


_SPLIT_SPECS = [
    pl.BlockSpec((_BN, DH), lambda i: (i, 0)),
    pl.BlockSpec((_BN, DH), lambda i: (_NB + i, 0)),
]


def _tc_post(parts, g2, dinv, b, ln_g, ln_b):
    def body(p0, p1, g0, g1, d_ref, b_ref, lg_ref, lb_ref, o_ref):
        r = _combine(p0, p1, g0, g1, d_ref, b_ref)
        mu = jnp.mean(r, axis=-1, keepdims=True)
        var = jnp.mean((r - mu) ** 2, axis=-1, keepdims=True)
        o_ref[...] = (r - mu) * lax.rsqrt(var + 1e-5) * lg_ref[...] + lb_ref[...]

    return pl.pallas_call(
        body,
        grid=(_NB,),
        in_specs=_SPLIT_SPECS + _SPLIT_SPECS + [
            pl.BlockSpec((_BN, 1), lambda i: (i, 0)),
            pl.BlockSpec((1, D), lambda i: (0, 0)),
            pl.BlockSpec((1, D), lambda i: (0, 0)),
            pl.BlockSpec((1, D), lambda i: (0, 0)),
        ],
        out_specs=pl.BlockSpec((_BN, D), lambda i: (i, 0)),
        out_shape=jax.ShapeDtypeStruct((N_PAD, D), jnp.float32),
    )(parts, parts, g2, g2, dinv, b.reshape(1, D), ln_g.reshape(1, D),
      ln_b.reshape(1, D))


def _tc_head(parts, g2, dinv, b, Wp1, bp1, Wp2, bp2):
    def body(p0, p1, g0, g1, d_ref, b_ref, w1_ref, b1_ref, w2_ref, b2_ref,
             o_ref):
        r = _combine(p0, p1, g0, g1, d_ref, b_ref)
        t = jnp.dot(r, w1_ref[...], preferred_element_type=jnp.float32)
        t = t + b1_ref[...]
        t = jnp.dot(t, w2_ref[...], preferred_element_type=jnp.float32)
        o_ref[...] = t + b2_ref[...]

    return pl.pallas_call(
        body,
        grid=(_NB,),
        in_specs=_SPLIT_SPECS + _SPLIT_SPECS + [
            pl.BlockSpec((_BN, 1), lambda i: (i, 0)),
            pl.BlockSpec((1, D), lambda i: (0, 0)),
            pl.BlockSpec((D, D), lambda i: (0, 0)),
            pl.BlockSpec((1, D), lambda i: (0, 0)),
            pl.BlockSpec((D, D), lambda i: (0, 0)),
            pl.BlockSpec((1, D), lambda i: (0, 0)),
        ],
        out_specs=pl.BlockSpec((_BN, D), lambda i: (i, 0)),
        out_shape=jax.ShapeDtypeStruct((N_PAD, D), jnp.float32),
    )(parts, parts, g2, g2, dinv, b.reshape(1, D), Wp1, bp1.reshape(1, D),
      Wp2, bp2.reshape(1, D))


def kernel(x, edge_index, batch, W1, b1, W2, b2, W3, b3,
           ln1_g, ln1_b, ln2_g, ln2_b, Wp1, bp1, Wp2, bp2):
    src, dst = edge_index[0], edge_index[1]
    pad = E_PAD - E
    padi = jnp.arange(pad, dtype=src.dtype)
    trash = N + padi % (N_PAD - N)
    src_p = jnp.concatenate([src, trash])
    srcoff = jnp.concatenate([src_p, src_p + N_PAD])
    dst2 = jnp.concatenate([dst, trash]).reshape(NS * CPT, CHUNK)

    x_p = jnp.pad(x, ((0, N_PAD - N), (0, 0)))

    degp = _sc_deg(dst2).reshape(NC, N_PAD, 1)
    dinv = _tc_dinv(degp)

    g1 = _tc_pre(x_p, W1, dinv)
    p1 = _sc_scatter(g1, srcoff, dst2)
    y1 = _tc_post(p1, g1, dinv, b1, ln1_g, ln1_b)

    g2 = _tc_pre(y1, W2, dinv)
    p2 = _sc_scatter(g2, srcoff, dst2)
    y2 = _tc_post(p2, g2, dinv, b2, ln2_g, ln2_b)

    g3 = _tc_pre(y2, W3, dinv)
    p3 = _sc_scatter(g3, srcoff, dst2)
    out = _tc_head(p3, g3, dinv, b3, Wp1, bp1, Wp2, bp2)
    return out[:N]


# speedup vs baseline: 22.0034x; 1.3069x over previous
import functools

import jax
import jax.numpy as jnp
from jax import lax
from jax.experimental import pallas as pl
from jax.experimental.pallas import tpu as pltpu
from jax.experimental.pallas import tpu_sc as plsc

N = 10000
E = 320000
D = 128
DH = 64
N_PAD = 10240
NC, NS = 2, 16
NW = NC * NS
CHUNK = 128
CPT = 160
E_PAD = NS * CPT * CHUNK
EPT = CPT * CHUNK
CPT_DEG = E_PAD // (NW * CHUNK)
ROWS_PT = N_PAD // NS
_B = 4
_NBATCH = CPT // _B


@functools.cache
def _mesh():
    return plsc.VectorSubcoreMesh(
        core_axis_name="c", subcore_axis_name="s",
        num_cores=NC, num_subcores=NS)


def _sc_deg(dst2):
    @functools.partial(
        pl.kernel,
        mesh=_mesh(),
        out_type=jax.ShapeDtypeStruct((NC * N_PAD,), jnp.float32),
        scratch_types=[
            pltpu.VMEM((CPT_DEG, CHUNK), jnp.int32),
            pltpu.VMEM((CHUNK,), jnp.float32),
            pltpu.VMEM((ROWS_PT,), jnp.float32),
            pltpu.VMEM_SHARED((N_PAD,), jnp.float32),
        ],
    )
    def k(dst_hbm, out_hbm, id2_v, ones_v, zbuf_v, acc_sh):
        c = lax.axis_index("c")
        s = lax.axis_index("s")
        wid = c * NS + s

        pltpu.sync_copy(dst_hbm.at[pl.ds(wid * CPT_DEG, CPT_DEG)], id2_v)

        @pl.loop(0, CHUNK // 16)
        def _(i):
            ones_v[pl.ds(i * 16, 16)] = jnp.ones((16,), jnp.float32)

        @pl.loop(0, ROWS_PT // 16)
        def _(i):
            zbuf_v[pl.ds(i * 16, 16)] = jnp.zeros((16,), jnp.float32)

        pltpu.sync_copy(zbuf_v, acc_sh.at[pl.ds(s * ROWS_PT, ROWS_PT)])
        plsc.subcore_barrier()

        @pl.loop(0, CPT_DEG)
        def _(j):
            pltpu.sync_copy(ones_v, acc_sh.at[id2_v.at[j]], add=True)

        plsc.subcore_barrier()
        pltpu.sync_copy(acc_sh.at[pl.ds(s * ROWS_PT, ROWS_PT)],
                        out_hbm.at[pl.ds(c * N_PAD + s * ROWS_PT, ROWS_PT)])

    return k(dst2)


def _sc_scatter(g2, srcoff, dst2):
    @functools.partial(
        pl.kernel,
        mesh=_mesh(),
        out_type=jax.ShapeDtypeStruct((NC * N_PAD, DH), jnp.float32),
        compiler_params=pltpu.CompilerParams(use_tc_tiling_on_sc=False),
        scratch_types=[
            pltpu.VMEM((CPT, CHUNK), jnp.int32),
            pltpu.VMEM((_B, CHUNK, DH), jnp.float32),
            pltpu.VMEM((_B, CHUNK, DH), jnp.float32),
            [pltpu.VMEM((CHUNK,), jnp.int32)] * (2 * _B),
            pltpu.VMEM_SHARED((N_PAD, DH), jnp.float32),
            pltpu.SemaphoreType.DMA,
            pltpu.SemaphoreType.DMA,
            pltpu.SemaphoreType.DMA,
        ],
    )
    def k(g_hbm, src_hbm, dst_hbm, out_hbm, id2_v, rows_a, rows_b,
          sbufs_flat, acc_sh, semg, sems, semi):
        halves = (rows_a, rows_b)
        sbufs = (sbufs_flat[:_B], sbufs_flat[_B:])
        c = lax.axis_index("c")
        s = lax.axis_index("s")

        pltpu.sync_copy(dst_hbm.at[pl.ds(s * CPT, CPT)], id2_v)

        @pl.loop(0, CHUNK)
        def _(i):
            for l in range(DH // 16):
                rows_a[0, i, pl.ds(l * 16, 16)] = jnp.zeros(
                    (16,), jnp.float32)

        for r in range(ROWS_PT // CHUNK):
            pltpu.sync_copy(
                rows_a.at[0],
                acc_sh.at[pl.ds(s * ROWS_PT + r * CHUNK, CHUNK)])

        plsc.subcore_barrier()

        src_base = c * E_PAD + s * EPT

        def fire_i(t, h):
            for i in range(_B):
                pltpu.async_copy(
                    src_hbm.at[pl.ds(src_base + (t * _B + i) * CHUNK,
                                     CHUNK)], sbufs[h][i], semi)

        def drain_i(t, h):
            for i in range(_B):
                pltpu.make_async_copy(
                    src_hbm.at[pl.ds(src_base + (t * _B + i) * CHUNK,
                                     CHUNK)], sbufs[h][i], semi).wait()

        def fire_g(t, h):
            for i in range(_B):
                pltpu.async_copy(g_hbm.at[sbufs[h][i]], halves[h].at[i],
                                 semg)

        def drain_g(t, h):
            for i in range(_B):
                pltpu.make_async_copy(
                    g_hbm.at[sbufs[h][i]], halves[h].at[i], semg).wait()

        def fire_s(t, h):
            for i in range(_B):
                pltpu.async_copy(
                    halves[h].at[i], acc_sh.at[id2_v.at[t * _B + i]], sems,
                    add=True)

        def drain_s(t, h):
            for i in range(_B):
                pltpu.make_async_copy(
                    halves[h].at[i], acc_sh.at[id2_v.at[t * _B + i]],
                    sems).wait()

        fire_i(0, 0)
        fire_i(1, 1)
        drain_i(0, 0)
        fire_g(0, 0)

        @pl.loop(0, _NBATCH // 2)
        def _(u):
            for h in range(2):
                t = 2 * u + h
                drain_g(t, h)

                @pl.when(t + 2 < _NBATCH)
                def _():
                    fire_i(t + 2, h)

                @pl.when(t + 1 < _NBATCH)
                def _():
                    drain_i(t + 1, 1 - h)
                    fire_g(t + 1, 1 - h)

                fire_s(t, h)
                drain_s(t, h)

        plsc.subcore_barrier()
        pltpu.sync_copy(acc_sh.at[pl.ds(s * ROWS_PT, ROWS_PT)],
                        out_hbm.at[pl.ds(c * N_PAD + s * ROWS_PT, ROWS_PT)])

    return k(g2, srcoff, dst2)


_BN = 1024
_NB = N_PAD // _BN


def _tc_dinv(degp):
    def body(dp_ref, o_ref):
        i = pl.program_id(0)
        d = dp_ref[0] + dp_ref[1] + 1.0
        row = lax.broadcasted_iota(jnp.int32, (_BN, 1), 0) + i * _BN
        o_ref[...] = jnp.where(row < N, lax.rsqrt(d), 0.0)

    return pl.pallas_call(
        body,
        grid=(_NB,),
        in_specs=[pl.BlockSpec((NC, _BN, 1), lambda i: (0, i, 0))],
        out_specs=pl.BlockSpec((_BN, 1), lambda i: (i, 0)),
        out_shape=jax.ShapeDtypeStruct((N_PAD, 1), jnp.float32),
    )(degp)


def _tc_pre(y, W, dinv):
    Wsp = jnp.stack([W[:, :DH], W[:, DH:]], axis=0)

    def body(y_ref, w_ref, d_ref, o_ref):
        h = jnp.dot(y_ref[...], w_ref[0],
                    preferred_element_type=jnp.float32)
        o_ref[...] = h * d_ref[...]

    return pl.pallas_call(
        body,
        grid=(_NB, NC),
        in_specs=[
            pl.BlockSpec((_BN, D), lambda i, c: (i, 0)),
            pl.BlockSpec((1, D, DH), lambda i, c: (c, 0, 0)),
            pl.BlockSpec((_BN, 1), lambda i, c: (i, 0)),
        ],
        out_specs=pl.BlockSpec((_BN, DH), lambda i, c: (c * _NB + i, 0)),
        out_shape=jax.ShapeDtypeStruct((NC * N_PAD, DH), jnp.float32),
    )(y, Wsp, dinv)


def _combine(p0_ref, p1_ref, g0_ref, g1_ref, d_ref, b_ref):
    p = jnp.concatenate([p0_ref[...] + g0_ref[...],
                         p1_ref[...] + g1_ref[...]], axis=1)
    return jnp.maximum(p * d_ref[...] + b_ref[...], 0.0)


_SPLIT_SPECS = [
    pl.BlockSpec((_BN, DH), lambda i: (i, 0)),
    pl.BlockSpec((_BN, DH), lambda i: (_NB + i, 0)),
]


def _tc_post(parts, g2, dinv, b, ln_g, ln_b):
    def body(p0, p1, g0, g1, d_ref, b_ref, lg_ref, lb_ref, o_ref):
        r = _combine(p0, p1, g0, g1, d_ref, b_ref)
        mu = jnp.mean(r, axis=-1, keepdims=True)
        var = jnp.mean((r - mu) ** 2, axis=-1, keepdims=True)
        o_ref[...] = (r - mu) * lax.rsqrt(var + 1e-5) * lg_ref[...] + lb_ref[...]

    return pl.pallas_call(
        body,
        grid=(_NB,),
        in_specs=_SPLIT_SPECS + _SPLIT_SPECS + [
            pl.BlockSpec((_BN, 1), lambda i: (i, 0)),
            pl.BlockSpec((1, D), lambda i: (0, 0)),
            pl.BlockSpec((1, D), lambda i: (0, 0)),
            pl.BlockSpec((1, D), lambda i: (0, 0)),
        ],
        out_specs=pl.BlockSpec((_BN, D), lambda i: (i, 0)),
        out_shape=jax.ShapeDtypeStruct((N_PAD, D), jnp.float32),
    )(parts, parts, g2, g2, dinv, b.reshape(1, D), ln_g.reshape(1, D),
      ln_b.reshape(1, D))


def _tc_head(parts, g2, dinv, b, Wp1, bp1, Wp2, bp2):
    def body(p0, p1, g0, g1, d_ref, b_ref, w1_ref, b1_ref, w2_ref, b2_ref,
             o_ref):
        r = _combine(p0, p1, g0, g1, d_ref, b_ref)
        t = jnp.dot(r, w1_ref[...], preferred_element_type=jnp.float32)
        t = t + b1_ref[...]
        t = jnp.dot(t, w2_ref[...], preferred_element_type=jnp.float32)
        o_ref[...] = t + b2_ref[...]

    return pl.pallas_call(
        body,
        grid=(_NB,),
        in_specs=_SPLIT_SPECS + _SPLIT_SPECS + [
            pl.BlockSpec((_BN, 1), lambda i: (i, 0)),
            pl.BlockSpec((1, D), lambda i: (0, 0)),
            pl.BlockSpec((D, D), lambda i: (0, 0)),
            pl.BlockSpec((1, D), lambda i: (0, 0)),
            pl.BlockSpec((D, D), lambda i: (0, 0)),
            pl.BlockSpec((1, D), lambda i: (0, 0)),
        ],
        out_specs=pl.BlockSpec((_BN, D), lambda i: (i, 0)),
        out_shape=jax.ShapeDtypeStruct((N_PAD, D), jnp.float32),
    )(parts, parts, g2, g2, dinv, b.reshape(1, D), Wp1, bp1.reshape(1, D),
      Wp2, bp2.reshape(1, D))


def kernel(x, edge_index, batch, W1, b1, W2, b2, W3, b3,
           ln1_g, ln1_b, ln2_g, ln2_b, Wp1, bp1, Wp2, bp2):
    src, dst = edge_index[0], edge_index[1]
    pad = E_PAD - E
    padi = jnp.arange(pad, dtype=src.dtype)
    trash = N + padi % (N_PAD - N)
    src_p = jnp.concatenate([src, trash])
    srcoff = jnp.concatenate([src_p, src_p + N_PAD])
    dst2 = jnp.concatenate([dst, trash]).reshape(NS * CPT, CHUNK)

    x_p = jnp.pad(x, ((0, N_PAD - N), (0, 0)))

    degp = _sc_deg(dst2).reshape(NC, N_PAD, 1)
    dinv = _tc_dinv(degp)

    g1 = _tc_pre(x_p, W1, dinv)
    p1 = _sc_scatter(g1, srcoff, dst2)
    y1 = _tc_post(p1, g1, dinv, b1, ln1_g, ln1_b)

    g2 = _tc_pre(y1, W2, dinv)
    p2 = _sc_scatter(g2, srcoff, dst2)
    y2 = _tc_post(p2, g2, dinv, b2, ln2_g, ln2_b)

    g3 = _tc_pre(y2, W3, dinv)
    p3 = _sc_scatter(g3, srcoff, dst2)
    out = _tc_head(p3, g3, dinv, b3, Wp1, bp1, Wp2, bp2)
    return out[:N]


# speedup vs baseline: 22.5694x; 1.0257x over previous
import functools

import jax
import jax.numpy as jnp
from jax import lax
from jax.experimental import pallas as pl
from jax.experimental.pallas import tpu as pltpu
from jax.experimental.pallas import tpu_sc as plsc

N = 10000
E = 320000
D = 128
DH = 64
N_PAD = 10240
NC, NS = 2, 16
NW = NC * NS
CHUNK = 128
CPT = 160
E_PAD = NS * CPT * CHUNK
EPT = CPT * CHUNK
CPT_DEG = E_PAD // (NW * CHUNK)
ROWS_PT = N_PAD // NS
_B = 4
_NBATCH = CPT // _B


@functools.cache
def _mesh():
    return plsc.VectorSubcoreMesh(
        core_axis_name="c", subcore_axis_name="s",
        num_cores=NC, num_subcores=NS)


def _sc_deg(dst2):
    @functools.partial(
        pl.kernel,
        mesh=_mesh(),
        out_type=jax.ShapeDtypeStruct((NC * N_PAD,), jnp.float32),
        scratch_types=[
            pltpu.VMEM((CPT_DEG, CHUNK), jnp.int32),
            pltpu.VMEM((CHUNK,), jnp.float32),
            pltpu.VMEM((ROWS_PT,), jnp.float32),
            pltpu.VMEM_SHARED((N_PAD,), jnp.float32),
        ],
    )
    def k(dst_hbm, out_hbm, id2_v, ones_v, zbuf_v, acc_sh):
        c = lax.axis_index("c")
        s = lax.axis_index("s")
        wid = c * NS + s

        pltpu.sync_copy(dst_hbm.at[pl.ds(wid * CPT_DEG, CPT_DEG)], id2_v)

        @pl.loop(0, CHUNK // 16)
        def _(i):
            ones_v[pl.ds(i * 16, 16)] = jnp.ones((16,), jnp.float32)

        @pl.loop(0, ROWS_PT // 16)
        def _(i):
            zbuf_v[pl.ds(i * 16, 16)] = jnp.zeros((16,), jnp.float32)

        pltpu.sync_copy(zbuf_v, acc_sh.at[pl.ds(s * ROWS_PT, ROWS_PT)])
        plsc.subcore_barrier()

        @pl.loop(0, CPT_DEG)
        def _(j):
            pltpu.sync_copy(ones_v, acc_sh.at[id2_v.at[j]], add=True)

        plsc.subcore_barrier()
        pltpu.sync_copy(acc_sh.at[pl.ds(s * ROWS_PT, ROWS_PT)],
                        out_hbm.at[pl.ds(c * N_PAD + s * ROWS_PT, ROWS_PT)])

    return k(dst2)


def _sc_scatter(g2, srcoff, dst2):
    @functools.partial(
        pl.kernel,
        mesh=_mesh(),
        out_type=jax.ShapeDtypeStruct((NC * N_PAD, DH), jnp.float32),
        compiler_params=pltpu.CompilerParams(use_tc_tiling_on_sc=False),
        scratch_types=[
            pltpu.VMEM((CPT, CHUNK), jnp.int32),
            pltpu.VMEM((_B, CHUNK, DH), jnp.float32),
            pltpu.VMEM((_B, CHUNK, DH), jnp.float32),
            [pltpu.VMEM((CHUNK,), jnp.int32)] * (2 * _B),
            pltpu.VMEM_SHARED((N_PAD, DH), jnp.float32),
            pltpu.SemaphoreType.DMA,
            pltpu.SemaphoreType.DMA,
            pltpu.SemaphoreType.DMA,
        ],
    )
    def k(g_hbm, src_hbm, dst_hbm, out_hbm, id2_v, rows_a, rows_b,
          sbufs_flat, acc_sh, semg, sems, semi):
        halves = (rows_a, rows_b)
        sbufs = (sbufs_flat[:_B], sbufs_flat[_B:])
        c = lax.axis_index("c")
        s = lax.axis_index("s")

        pltpu.sync_copy(dst_hbm.at[pl.ds(s * CPT, CPT)], id2_v)

        @pl.loop(0, CHUNK)
        def _(i):
            for l in range(DH // 16):
                rows_a[0, i, pl.ds(l * 16, 16)] = jnp.zeros(
                    (16,), jnp.float32)

        for r in range(ROWS_PT // CHUNK):
            pltpu.sync_copy(
                rows_a.at[0],
                acc_sh.at[pl.ds(s * ROWS_PT + r * CHUNK, CHUNK)])

        plsc.subcore_barrier()

        src_base = c * E_PAD + s * EPT

        def fire_i(t, h):
            for i in range(_B):
                pltpu.async_copy(
                    src_hbm.at[pl.ds(src_base + (t * _B + i) * CHUNK,
                                     CHUNK)], sbufs[h][i], semi)

        def drain_i(t, h):
            for i in range(_B):
                pltpu.make_async_copy(
                    src_hbm.at[pl.ds(src_base + (t * _B + i) * CHUNK,
                                     CHUNK)], sbufs[h][i], semi).wait()

        def fire_g(t, h):
            for i in range(_B):
                pltpu.async_copy(g_hbm.at[sbufs[h][i]], halves[h].at[i],
                                 semg)

        def drain_g(t, h):
            for i in range(_B):
                pltpu.make_async_copy(
                    g_hbm.at[sbufs[h][i]], halves[h].at[i], semg).wait()

        def fire_s(t, h):
            for i in range(_B):
                pltpu.async_copy(
                    halves[h].at[i], acc_sh.at[id2_v.at[t * _B + i]], sems,
                    add=True)

        def drain_s(t, h):
            for i in range(_B):
                pltpu.make_async_copy(
                    halves[h].at[i], acc_sh.at[id2_v.at[t * _B + i]],
                    sems).wait()

        fire_i(0, 0)
        fire_i(1, 1)
        drain_i(0, 0)
        fire_g(0, 0)

        @pl.loop(0, _NBATCH // 2)
        def _(u):
            for h in range(2):
                t = 2 * u + h
                drain_g(t, h)

                @pl.when(t + 2 < _NBATCH)
                def _():
                    fire_i(t + 2, h)

                @pl.when(t + 1 < _NBATCH)
                def _():
                    drain_i(t + 1, 1 - h)
                    fire_g(t + 1, 1 - h)

                fire_s(t, h)
                drain_s(t, h)

        plsc.subcore_barrier()
        pltpu.sync_copy(acc_sh.at[pl.ds(s * ROWS_PT, ROWS_PT)],
                        out_hbm.at[pl.ds(c * N_PAD + s * ROWS_PT, ROWS_PT)])

    return k(g2, srcoff, dst2)


_BN = 1024
_NB = N_PAD // _BN


def _dinv_block(dp_ref, i):
    d = dp_ref[0] + dp_ref[1] + 1.0
    row = lax.broadcasted_iota(jnp.int32, (_BN, 1), 0) + i * _BN
    return jnp.where(row < N, lax.rsqrt(d), 0.0)


_DEG_SPEC2 = pl.BlockSpec((NC, _BN, 1), lambda i, c: (0, i, 0))
_DEG_SPEC1 = pl.BlockSpec((NC, _BN, 1), lambda i: (0, i, 0))


def _tc_pre(y, W, degp):
    Wsp = jnp.stack([W[:, :DH], W[:, DH:]], axis=0)

    def body(y_ref, w_ref, dp_ref, o_ref):
        h = jnp.dot(y_ref[...], w_ref[0],
                    preferred_element_type=jnp.float32)
        o_ref[...] = h * _dinv_block(dp_ref, pl.program_id(0))

    return pl.pallas_call(
        body,
        grid=(_NB, NC),
        in_specs=[
            pl.BlockSpec((_BN, D), lambda i, c: (i, 0)),
            pl.BlockSpec((1, D, DH), lambda i, c: (c, 0, 0)),
            _DEG_SPEC2,
        ],
        out_specs=pl.BlockSpec((_BN, DH), lambda i, c: (c * _NB + i, 0)),
        out_shape=jax.ShapeDtypeStruct((NC * N_PAD, DH), jnp.float32),
    )(y, Wsp, degp)


def _combine(p0_ref, p1_ref, g0_ref, g1_ref, dinv, b_ref):
    p = jnp.concatenate([p0_ref[...] + g0_ref[...],
                         p1_ref[...] + g1_ref[...]], axis=1)
    return jnp.maximum(p * dinv + b_ref[...], 0.0)


_SPLIT_SPECS = [
    pl.BlockSpec((_BN, DH), lambda i: (i, 0)),
    pl.BlockSpec((_BN, DH), lambda i: (_NB + i, 0)),
]
_SPLIT_SPECS2 = [
    pl.BlockSpec((_BN, DH), lambda i, c: (i, 0)),
    pl.BlockSpec((_BN, DH), lambda i, c: (_NB + i, 0)),
]


def _tc_mid(parts, g2, degp, b, ln_g, ln_b, Wn):
    Wsp = jnp.stack([Wn[:, :DH], Wn[:, DH:]], axis=0)

    def body(p0, p1, g0, g1, dp_ref, b_ref, lg_ref, lb_ref, w_ref, o_ref):
        dinv = _dinv_block(dp_ref, pl.program_id(0))
        r = _combine(p0, p1, g0, g1, dinv, b_ref)
        mu = jnp.mean(r, axis=-1, keepdims=True)
        var = jnp.mean((r - mu) ** 2, axis=-1, keepdims=True)
        y = (r - mu) * lax.rsqrt(var + 1e-5) * lg_ref[...] + lb_ref[...]
        h = jnp.dot(y, w_ref[0], preferred_element_type=jnp.float32)
        o_ref[...] = h * dinv

    return pl.pallas_call(
        body,
        grid=(_NB, NC),
        in_specs=_SPLIT_SPECS2 + _SPLIT_SPECS2 + [
            _DEG_SPEC2,
            pl.BlockSpec((1, D), lambda i, c: (0, 0)),
            pl.BlockSpec((1, D), lambda i, c: (0, 0)),
            pl.BlockSpec((1, D), lambda i, c: (0, 0)),
            pl.BlockSpec((1, D, DH), lambda i, c: (c, 0, 0)),
        ],
        out_specs=pl.BlockSpec((_BN, DH), lambda i, c: (c * _NB + i, 0)),
        out_shape=jax.ShapeDtypeStruct((NC * N_PAD, DH), jnp.float32),
    )(parts, parts, g2, g2, degp, b.reshape(1, D), ln_g.reshape(1, D),
      ln_b.reshape(1, D), Wsp)


def _tc_head(parts, g2, degp, b, Wp1, bp1, Wp2, bp2):
    def body(p0, p1, g0, g1, dp_ref, b_ref, w1_ref, b1_ref, w2_ref, b2_ref,
             o_ref):
        dinv = _dinv_block(dp_ref, pl.program_id(0))
        r = _combine(p0, p1, g0, g1, dinv, b_ref)
        t = jnp.dot(r, w1_ref[...], preferred_element_type=jnp.float32)
        t = t + b1_ref[...]
        t = jnp.dot(t, w2_ref[...], preferred_element_type=jnp.float32)
        o_ref[...] = t + b2_ref[...]

    return pl.pallas_call(
        body,
        grid=(_NB,),
        in_specs=_SPLIT_SPECS + _SPLIT_SPECS + [
            _DEG_SPEC1,
            pl.BlockSpec((1, D), lambda i: (0, 0)),
            pl.BlockSpec((D, D), lambda i: (0, 0)),
            pl.BlockSpec((1, D), lambda i: (0, 0)),
            pl.BlockSpec((D, D), lambda i: (0, 0)),
            pl.BlockSpec((1, D), lambda i: (0, 0)),
        ],
        out_specs=pl.BlockSpec((_BN, D), lambda i: (i, 0)),
        out_shape=jax.ShapeDtypeStruct((N_PAD, D), jnp.float32),
    )(parts, parts, g2, g2, degp, b.reshape(1, D), Wp1, bp1.reshape(1, D),
      Wp2, bp2.reshape(1, D))


def kernel(x, edge_index, batch, W1, b1, W2, b2, W3, b3,
           ln1_g, ln1_b, ln2_g, ln2_b, Wp1, bp1, Wp2, bp2):
    src, dst = edge_index[0], edge_index[1]
    pad = E_PAD - E
    padi = jnp.arange(pad, dtype=src.dtype)
    trash = N + padi % (N_PAD - N)
    src_p = jnp.concatenate([src, trash])
    srcoff = jnp.concatenate([src_p, src_p + N_PAD])
    dst2 = jnp.concatenate([dst, trash]).reshape(NS * CPT, CHUNK)

    x_p = jnp.pad(x, ((0, N_PAD - N), (0, 0)))

    degp = _sc_deg(dst2).reshape(NC, N_PAD, 1)

    g1 = _tc_pre(x_p, W1, degp)
    p1 = _sc_scatter(g1, srcoff, dst2)
    g2 = _tc_mid(p1, g1, degp, b1, ln1_g, ln1_b, W2)
    p2 = _sc_scatter(g2, srcoff, dst2)
    g3 = _tc_mid(p2, g2, degp, b2, ln2_g, ln2_b, W3)
    p3 = _sc_scatter(g3, srcoff, dst2)
    out = _tc_head(p3, g3, degp, b3, Wp1, bp1, Wp2, bp2)
    return out[:N]
